# baseline (speedup 1.0000x reference)
import functools

import jax
import jax.numpy as jnp
from jax import lax
from jax.experimental import pallas as pl
from jax.experimental.pallas import tpu as pltpu
from jax.experimental.pallas import tpu_sc as plsc

B = 16384
NUM_NUMERIC = 13
N_CAT = 26
CARD = 100000
D = 32

NC = 2
NS = 16
NW = NC * NS
B_PER_W = B // NW
CB = 64
N_CHUNKS = B_PER_W // CB


def _tok_kernel(cat_t_hbm, tables_hbm, numeric_hbm, w_hbm, bias_hbm, cls_hbm,
                out_hbm,
                idx_v, rows_v, numtok_v, num_v, w_v, b_v, cls_v,
                gsem):
    wid = lax.axis_index("s") * NC + lax.axis_index("c")
    base = wid * B_PER_W

    pltpu.sync_copy(w_hbm, w_v)
    pltpu.sync_copy(bias_hbm, b_v)
    pltpu.sync_copy(cls_hbm, cls_v)

    cls_lo = cls_v[0, 0, pl.ds(0, 16)]
    cls_hi = cls_v[0, 0, pl.ds(16, 16)]

    def fill_cls(cb, carry):
        numtok_v[cb, 0, pl.ds(0, 16)] = cls_lo
        numtok_v[cb, 0, pl.ds(16, 16)] = cls_hi
        return carry
    lax.fori_loop(0, CB, fill_cls, 0, unroll=4)

    def chunk_body(c, carry):
        b0 = base + c * CB

        pltpu.sync_copy(cat_t_hbm.at[:, pl.ds(b0, CB)], idx_v)
        pltpu.sync_copy(numeric_hbm.at[pl.ds(b0, CB)], num_v)

        def fire(f, carry2):
            off = f * CARD
            for g in range(CB // 16):
                sl = pl.ds(g * 16, 16)
                idx_v[f, sl] = idx_v[f, sl] + off
            pltpu.async_copy(tables_hbm.at[idx_v.at[f]], rows_v.at[f], gsem)
            return carry2
        lax.fori_loop(0, N_CAT, fire, 0)

        for i in range(NUM_NUMERIC):
            w_lo = w_v[i, pl.ds(0, 16)]
            w_hi = w_v[i, pl.ds(16, 16)]
            bi_lo = b_v[i, pl.ds(0, 16)]
            bi_hi = b_v[i, pl.ds(16, 16)]

            def num_row(cb, carry3, i=i, w_lo=w_lo, w_hi=w_hi,
                        bi_lo=bi_lo, bi_hi=bi_hi):
                v = num_v[cb, pl.ds(0, 16)][i]
                numtok_v[cb, i + 1, pl.ds(0, 16)] = v * w_lo + bi_lo
                numtok_v[cb, i + 1, pl.ds(16, 16)] = v * w_hi + bi_hi
                return carry3
            lax.fori_loop(0, CB, num_row, 0, unroll=2)

        pltpu.sync_copy(numtok_v, out_hbm.at[pl.ds(b0, CB), pl.ds(0, 14)])

        def drain(f, carry2):
            pltpu.make_async_copy(tables_hbm.at[idx_v.at[f]], rows_v.at[f],
                                  gsem).wait()
            pltpu.sync_copy(rows_v.at[f],
                            out_hbm.at[pl.ds(b0, CB), 14 + f])
            return carry2
        lax.fori_loop(0, N_CAT, drain, 0)
        return carry

    lax.fori_loop(0, N_CHUNKS, chunk_body, 0)


@jax.jit
def kernel(numeric, categorical, num_weight, num_bias, cat_tables, cls_token):
    cat_t = categorical.T
    tables = cat_tables.reshape(N_CAT * CARD, D)
    numeric_pad = jnp.pad(numeric, ((0, 0), (0, 16 - NUM_NUMERIC)))

    run = pl.kernel(
        _tok_kernel,
        out_type=jax.ShapeDtypeStruct((B, 1 + NUM_NUMERIC + N_CAT, D),
                                      jnp.float32),
        mesh=plsc.VectorSubcoreMesh(core_axis_name="c", subcore_axis_name="s"),
        compiler_params=pltpu.CompilerParams(use_tc_tiling_on_sc=False),
        scratch_types=[
            pltpu.VMEM((N_CAT, CB), jnp.int32),
            pltpu.VMEM((N_CAT, CB, D), jnp.float32),
            pltpu.VMEM((CB, 1 + NUM_NUMERIC, D), jnp.float32),
            pltpu.VMEM((CB, 16), jnp.float32),
            pltpu.VMEM((NUM_NUMERIC, D), jnp.float32),
            pltpu.VMEM((NUM_NUMERIC, D), jnp.float32),
            pltpu.VMEM((1, 1, D), jnp.float32),
            pltpu.SemaphoreType.DMA,
        ],
    )
    return run(cat_t, tables, numeric_pad, num_weight, num_bias, cls_token)


# speedup vs baseline: 1.4069x; 1.4069x over previous
import jax
import jax.numpy as jnp
from jax import lax
from jax.experimental import pallas as pl
from jax.experimental.pallas import tpu as pltpu
from jax.experimental.pallas import tpu_sc as plsc

B = 16384
NUM_NUMERIC = 13
N_CAT = 26
CARD = 100000
D = 32
N_TOK = 1 + NUM_NUMERIC + N_CAT

NC = 2
NS = 16
NW = NC * NS
B_PER_W = B // NW
NB = 64
N_NCHUNK = B_PER_W // NB


def _tok_kernel(cat_t_hbm, tbl_hbm, num_t_hbm, w_hbm, bias_hbm, cls_hbm,
                out_hbm,
                idx_v, num_v, numbuf_v, outbuf_a, outbuf_b, outbuf_c,
                outbuf_d, w_v, b_v, cls_v,
                gsem, osem, nsem):
    wid = lax.axis_index("s") * NC + lax.axis_index("c")
    base = wid * B_PER_W

    pltpu.sync_copy(w_hbm, w_v)
    pltpu.sync_copy(bias_hbm, b_v)
    pltpu.sync_copy(cls_hbm, cls_v)
    pltpu.sync_copy(cat_t_hbm.at[:, pl.ds(base, B_PER_W)], idx_v)
    pltpu.sync_copy(num_t_hbm.at[:, pl.ds(base, B_PER_W)], num_v)

    def splat_at(ref2d, i, j):
        isplat = jnp.full((16,), i, dtype=jnp.int32)
        jsplat = jnp.full((16,), j, dtype=jnp.int32)
        return plsc.load_gather(ref2d, [isplat, jsplat])

    def fire_field(f, outbuf_v):
        def fire_d(d, carry):
            pltpu.async_copy(tbl_hbm.at[f, d].at[idx_v.at[f]],
                             outbuf_v.at[d], gsem)
            return carry
        lax.fori_loop(0, D, fire_d, 0)

    def drain_field(f, outbuf_v):
        def wait_d(d, carry):
            pltpu.make_async_copy(tbl_hbm.at[f, d].at[idx_v.at[f]],
                                  outbuf_v.at[d], gsem).wait()
            return carry
        lax.fori_loop(0, D, wait_d, 0)

    bufs = [outbuf_a, outbuf_b, outbuf_c, outbuf_d]
    for f in range(3):
        fire_field(f, bufs[f])

    def nchunk(c, carry):
        b0 = base + c * NB

        def cls_d(d, carry2):
            v = splat_at(cls_v, 0, d)
            for g in range(NB // 16):
                numbuf_v[0, d, pl.ds(g * 16, 16)] = v
            return carry2
        lax.fori_loop(0, D, cls_d, 0)

        def num_i(i, carry2):
            def num_d(d, carry3):
                w_id = splat_at(w_v, i, d)
                b_id = splat_at(b_v, i, d)
                for g in range(NB // 16):
                    nv = num_v[i, pl.ds(c * NB + g * 16, 16)]
                    numbuf_v[i + 1, d, pl.ds(g * 16, 16)] = nv * w_id + b_id
                return carry3
            lax.fori_loop(0, D, num_d, 0)
            return carry2
        lax.fori_loop(0, NUM_NUMERIC, num_i, 0)

        pltpu.sync_copy(
            numbuf_v, out_hbm.at[pl.ds(0, 1 + NUM_NUMERIC), :, pl.ds(b0, NB)])
        return carry
    lax.fori_loop(0, N_NCHUNK, nchunk, 0)

    def out_slab(f):
        return out_hbm.at[14 + f, :, pl.ds(base, B_PER_W)]

    for f in range(N_CAT):
        if f >= 1:
            pltpu.make_async_copy(bufs[(f - 1) % 4], out_slab(f - 1),
                                  osem).wait()
        if f + 3 < N_CAT:
            fire_field(f + 3, bufs[(f + 3) % 4])
        drain_field(f, bufs[f % 4])
        pltpu.async_copy(bufs[f % 4], out_slab(f), osem)

    pltpu.make_async_copy(bufs[(N_CAT - 1) % 4], out_slab(N_CAT - 1),
                          osem).wait()


@jax.jit
def kernel(numeric, categorical, num_weight, num_bias, cat_tables, cls_token):
    cat_t = categorical.T
    num_t = numeric.T
    tbl_sw = jnp.swapaxes(cat_tables, 1, 2)

    run = pl.kernel(
        _tok_kernel,
        out_type=jax.ShapeDtypeStruct((N_TOK, D, B), jnp.float32),
        mesh=plsc.VectorSubcoreMesh(core_axis_name="c", subcore_axis_name="s"),
        compiler_params=pltpu.CompilerParams(use_tc_tiling_on_sc=False,
                                             needs_layout_passes=False),
        scratch_types=[
            pltpu.VMEM((N_CAT, B_PER_W), jnp.int32),
            pltpu.VMEM((NUM_NUMERIC, B_PER_W), jnp.float32),
            pltpu.VMEM((1 + NUM_NUMERIC, D, NB), jnp.float32),
            pltpu.VMEM((D, B_PER_W), jnp.float32),
            pltpu.VMEM((D, B_PER_W), jnp.float32),
            pltpu.VMEM((D, B_PER_W), jnp.float32),
            pltpu.VMEM((D, B_PER_W), jnp.float32),
            pltpu.VMEM((NUM_NUMERIC, D), jnp.float32),
            pltpu.VMEM((NUM_NUMERIC, D), jnp.float32),
            pltpu.VMEM((1, D), jnp.float32),
            pltpu.SemaphoreType.DMA,
            pltpu.SemaphoreType.DMA,
            pltpu.SemaphoreType.DMA,
        ],
    )
    out_p = run(cat_t, tbl_sw, num_t, num_weight, num_bias,
                cls_token.reshape(1, D))
    return out_p.transpose(2, 0, 1)
